# baseline (speedup 1.0000x reference)
import functools

import jax
import jax.numpy as jnp
from jax import lax
from jax.experimental import pallas as pl
from jax.experimental.pallas import tpu as pltpu
from jax.experimental.pallas import tpu_sc as plsc

_B = 1024
_CTX = 20
_D = 64
_LANES = 16
_BLK = 128
_NSLOT = 4


@functools.lru_cache(maxsize=None)
def _build_sc_mean(nc: int, ns: int):
    nw = nc * ns
    bpw = _B // nw
    n_widx = bpw * _CTX

    mesh = plsc.VectorSubcoreMesh(core_axis_name="c", subcore_axis_name="s")

    @functools.partial(
        pl.kernel,
        mesh=mesh,
        out_type=jax.ShapeDtypeStruct((_B, _D), jnp.float32),
        compiler_params=pltpu.CompilerParams(use_tc_tiling_on_sc=True,
                                             needs_layout_passes=False),
        scratch_types=[
            pltpu.VMEM((bpw,), jnp.int32),
            pltpu.VMEM((n_widx,), jnp.int32),
            pltpu.VMEM((_NSLOT, _D, _BLK), jnp.float32),
            pltpu.VMEM((bpw, _D), jnp.float32),
            pltpu.VMEM((n_widx, _D), jnp.float32),
            pltpu.VMEM((bpw, _D), jnp.float32),
            [pltpu.SemaphoreType.DMA] * _NSLOT,
            pltpu.SemaphoreType.DMA,
        ],
    )
    def sc_mean(doc_ids_hbm, words_hbm, doc_tabt_hbm, word_tab_hbm, out_hbm,
                didx_v, widx_v, blocks_v, drows_v, wrows_v, orows_v,
                dsems, wsem):
        wid = lax.axis_index("s") * nc + lax.axis_index("c")
        base = wid * bpw
        pltpu.sync_copy(doc_ids_hbm.at[pl.ds(base, bpw)], didx_v)
        pltpu.sync_copy(words_hbm.at[pl.ds(base * _CTX, n_widx)], widx_v)

        def enq_word(c, carry):
            vec = widx_v[pl.ds(c * _LANES, _LANES)]
            for k in range(_LANES):
                pltpu.async_copy(word_tab_hbm.at[pl.ds(vec[k], 1)],
                                 wrows_v.at[pl.ds(c * _LANES + k, 1)], wsem)
            return carry

        lax.fori_loop(0, n_widx // _LANES, enq_word, 0)

        dvecs = [didx_v[pl.ds(c * _LANES, _LANES)]
                 for c in range(bpw // _LANES)]

        def enq_doc(d):
            idx = dvecs[d // _LANES][d % _LANES]
            col0 = pl.multiple_of((idx >> 7) << 7, _BLK)
            pltpu.async_copy(doc_tabt_hbm.at[:, pl.ds(col0, _BLK)],
                             blocks_v.at[d % _NSLOT], dsems[d % _NSLOT])

        def extract_doc(d):
            slot = d % _NSLOT
            pltpu.make_async_copy(doc_tabt_hbm.at[:, pl.ds(0, _BLK)],
                                  blocks_v.at[slot], dsems[slot]).wait()
            idx = dvecs[d // _LANES][d % _LANES]
            lane = jnp.broadcast_to(idx & 127, (_LANES,))
            for fc in range(_D // _LANES):
                rows = lax.iota(jnp.int32, _LANES) + fc * _LANES
                vals = plsc.load_gather(blocks_v.at[slot], [rows, lane])
                drows_v[d, pl.ds(fc * _LANES, _LANES)] = vals

        for d in range(_NSLOT):
            enq_doc(d)
        for d in range(_NSLOT, bpw):
            extract_doc(d - _NSLOT)
            enq_doc(d)
        for d in range(bpw - _NSLOT, bpw):
            extract_doc(d)

        pltpu.make_async_copy(word_tab_hbm.at[pl.ds(0, n_widx)], wrows_v,
                              wsem).wait()

        scale = jnp.float32(1.0 / (_CTX + 1))

        def body(i, carry):
            for c in range(_D // _LANES):
                sl = pl.ds(c * _LANES, _LANES)
                acc = drows_v[i, sl]
                for j in range(_CTX):
                    acc = acc + wrows_v[i * _CTX + j, sl]
                orows_v[i, sl] = acc * scale
            return carry

        lax.fori_loop(0, bpw, body, 0)
        pltpu.sync_copy(orows_v, out_hbm.at[pl.ds(base, bpw)])

    return sc_mean, nw


def _mm_body_t(wt_ref, meant_ref, b_ref, out_ref):
    acc = jax.lax.dot_general(
        wt_ref[...], meant_ref[...], (((0,), (0,)), ((), ())),
        preferred_element_type=jnp.float32,
    )
    out_ref[...] = acc + jnp.transpose(b_ref[...])


def _project_t(meant, Wt, b, tile_v: int = 2048):
    v = Wt.shape[1]
    grid = (pl.cdiv(v, tile_v),)
    return pl.pallas_call(
        _mm_body_t,
        grid=grid,
        in_specs=[
            pl.BlockSpec((_D, tile_v), lambda j: (0, j)),
            pl.BlockSpec((_D, _B), lambda j: (0, 0)),
            pl.BlockSpec((1, tile_v), lambda j: (0, j)),
        ],
        out_specs=pl.BlockSpec((tile_v, _B), lambda j: (j, 0)),
        out_shape=jax.ShapeDtypeStruct((v, _B), jnp.float32),
        compiler_params=pltpu.CompilerParams(
            dimension_semantics=("parallel",),
        ),
    )(Wt, meant, b.reshape(1, v))


def kernel(doc_ids, context_words, doc_table, word_table, W, b):
    info = plsc.get_sparse_core_info()
    sc_mean, nw = _build_sc_mean(info.num_cores, info.num_subcores)
    doc_ids32 = doc_ids.astype(jnp.int32)
    words = context_words.astype(jnp.int32).reshape(-1)
    mean_vec = sc_mean(doc_ids32, words, doc_table.T, word_table)
    return _project_t(mean_vec.T, W.T, b).T


# speedup vs baseline: 1.0047x; 1.0047x over previous
import functools

import jax
import jax.numpy as jnp
from jax import lax
from jax.experimental import pallas as pl
from jax.experimental.pallas import tpu as pltpu
from jax.experimental.pallas import tpu_sc as plsc

_B = 1024
_CTX = 20
_D = 64
_LANES = 16
_BLK = 128
_NSLOT = 4


@functools.lru_cache(maxsize=None)
def _build_sc_mean(nc: int, ns: int):
    nw = nc * ns
    bpw = _B // nw
    n_widx = bpw * _CTX

    mesh = plsc.VectorSubcoreMesh(core_axis_name="c", subcore_axis_name="s")

    @functools.partial(
        pl.kernel,
        mesh=mesh,
        out_type=jax.ShapeDtypeStruct((_B, _D), jnp.float32),
        compiler_params=pltpu.CompilerParams(use_tc_tiling_on_sc=True,
                                             needs_layout_passes=False),
        scratch_types=[
            pltpu.VMEM((bpw,), jnp.int32),
            pltpu.VMEM((n_widx,), jnp.int32),
            pltpu.VMEM((_NSLOT, _D, _BLK), jnp.float32),
            pltpu.VMEM((bpw, _D), jnp.float32),
            pltpu.VMEM((n_widx, _D), jnp.float32),
            pltpu.VMEM((bpw, _D), jnp.float32),
            [pltpu.SemaphoreType.DMA] * _NSLOT,
            pltpu.SemaphoreType.DMA,
        ],
    )
    def sc_mean(doc_ids_hbm, words_hbm, doc_tabt_hbm, word_tab_hbm, out_hbm,
                didx_v, widx_v, blocks_v, drows_v, wrows_v, orows_v,
                dsems, wsem):
        wid = lax.axis_index("s") * nc + lax.axis_index("c")
        base = wid * bpw
        pltpu.sync_copy(doc_ids_hbm.at[pl.ds(base, bpw)], didx_v)
        pltpu.sync_copy(words_hbm.at[pl.ds(base * _CTX, n_widx)], widx_v)

        def enq_word(c, carry):
            vec = widx_v[pl.ds(c * _LANES, _LANES)]
            for k in range(_LANES):
                pltpu.async_copy(word_tab_hbm.at[pl.ds(vec[k], 1)],
                                 wrows_v.at[pl.ds(c * _LANES + k, 1)], wsem)
            return carry

        lax.fori_loop(0, n_widx // _LANES, enq_word, 0)

        dvecs = [didx_v[pl.ds(c * _LANES, _LANES)]
                 for c in range(bpw // _LANES)]

        def enq_doc(d):
            idx = dvecs[d // _LANES][d % _LANES]
            col0 = pl.multiple_of((idx >> 7) << 7, _BLK)
            pltpu.async_copy(doc_tabt_hbm.at[:, pl.ds(col0, _BLK)],
                             blocks_v.at[d % _NSLOT], dsems[d % _NSLOT])

        def extract_doc(d):
            slot = d % _NSLOT
            pltpu.make_async_copy(doc_tabt_hbm.at[:, pl.ds(0, _BLK)],
                                  blocks_v.at[slot], dsems[slot]).wait()
            idx = dvecs[d // _LANES][d % _LANES]
            lane = jnp.broadcast_to(idx & 127, (_LANES,))
            for fc in range(_D // _LANES):
                rows = lax.iota(jnp.int32, _LANES) + fc * _LANES
                vals = plsc.load_gather(blocks_v.at[slot], [rows, lane])
                drows_v[d, pl.ds(fc * _LANES, _LANES)] = vals

        for d in range(_NSLOT):
            enq_doc(d)
        for d in range(_NSLOT, bpw):
            extract_doc(d - _NSLOT)
            enq_doc(d)
        for d in range(bpw - _NSLOT, bpw):
            extract_doc(d)

        pltpu.make_async_copy(word_tab_hbm.at[pl.ds(0, n_widx)], wrows_v,
                              wsem).wait()

        scale = jnp.float32(1.0 / (_CTX + 1))

        def body(i, carry):
            for c in range(_D // _LANES):
                sl = pl.ds(c * _LANES, _LANES)
                acc = drows_v[i, sl]
                for j in range(_CTX):
                    acc = acc + wrows_v[i * _CTX + j, sl]
                orows_v[i, sl] = acc * scale
            return carry

        lax.fori_loop(0, bpw, body, 0)
        pltpu.sync_copy(orows_v, out_hbm.at[pl.ds(base, bpw)])

    return sc_mean, nw


def _mm_body_t(wt_ref, meant_ref, b_ref, out_ref):
    acc = jax.lax.dot_general(
        wt_ref[...], meant_ref[...], (((0,), (0,)), ((), ())),
        preferred_element_type=jnp.float32,
    )
    out_ref[...] = acc + jnp.transpose(b_ref[...])


def _project_t(meant, Wt, b, tile_v: int = 4096):
    v = Wt.shape[1]
    grid = (pl.cdiv(v, tile_v),)
    return pl.pallas_call(
        _mm_body_t,
        grid=grid,
        in_specs=[
            pl.BlockSpec((_D, tile_v), lambda j: (0, j)),
            pl.BlockSpec((_D, _B), lambda j: (0, 0)),
            pl.BlockSpec((1, tile_v), lambda j: (0, j)),
        ],
        out_specs=pl.BlockSpec((tile_v, _B), lambda j: (j, 0)),
        out_shape=jax.ShapeDtypeStruct((v, _B), jnp.float32),
        compiler_params=pltpu.CompilerParams(
            dimension_semantics=("parallel",),
        ),
    )(Wt, meant, b.reshape(1, v))


def kernel(doc_ids, context_words, doc_table, word_table, W, b):
    info = plsc.get_sparse_core_info()
    sc_mean, nw = _build_sc_mean(info.num_cores, info.num_subcores)
    doc_ids32 = doc_ids.astype(jnp.int32)
    words = context_words.astype(jnp.int32).reshape(-1)
    mean_vec = sc_mean(doc_ids32, words, doc_table.T, word_table)
    return _project_t(mean_vec.T, W.T, b).T


# speedup vs baseline: 1.0057x; 1.0010x over previous
import functools

import jax
import jax.numpy as jnp
from jax import lax
from jax.experimental import pallas as pl
from jax.experimental.pallas import tpu as pltpu
from jax.experimental.pallas import tpu_sc as plsc

_B = 1024
_CTX = 20
_D = 64
_LANES = 16
_BLK = 128
_NSLOT = 4


@functools.lru_cache(maxsize=None)
def _build_sc_mean(nc: int, ns: int):
    nw = nc * ns
    bpw = _B // nw
    n_widx = bpw * _CTX

    mesh = plsc.VectorSubcoreMesh(core_axis_name="c", subcore_axis_name="s")

    @functools.partial(
        pl.kernel,
        mesh=mesh,
        out_type=jax.ShapeDtypeStruct((_B, _D), jnp.float32),
        compiler_params=pltpu.CompilerParams(use_tc_tiling_on_sc=True,
                                             needs_layout_passes=False),
        scratch_types=[
            pltpu.VMEM((bpw,), jnp.int32),
            pltpu.VMEM((n_widx,), jnp.int32),
            pltpu.VMEM((_NSLOT, _D, _BLK), jnp.float32),
            pltpu.VMEM((bpw, _D), jnp.float32),
            pltpu.VMEM((n_widx, _D), jnp.float32),
            pltpu.VMEM((bpw, _D), jnp.float32),
            [pltpu.SemaphoreType.DMA] * _NSLOT,
            pltpu.SemaphoreType.DMA,
        ],
    )
    def sc_mean(doc_ids_hbm, words_hbm, doc_tabt_hbm, word_tab_hbm, out_hbm,
                didx_v, widx_v, blocks_v, drows_v, wrows_v, orows_v,
                dsems, wsem):
        wid = lax.axis_index("s") * nc + lax.axis_index("c")
        base = wid * bpw
        pltpu.sync_copy(doc_ids_hbm.at[pl.ds(base, bpw)], didx_v)
        pltpu.sync_copy(words_hbm.at[pl.ds(base * _CTX, n_widx)], widx_v)

        dvecs = [didx_v[pl.ds(c * _LANES, _LANES)]
                 for c in range(bpw // _LANES)]

        def enq_doc(d):
            idx = dvecs[d // _LANES][d % _LANES]
            col0 = pl.multiple_of((idx >> 7) << 7, _BLK)
            pltpu.async_copy(doc_tabt_hbm.at[:, pl.ds(col0, _BLK)],
                             blocks_v.at[d % _NSLOT], dsems[d % _NSLOT])

        def extract_doc(d):
            slot = d % _NSLOT
            pltpu.make_async_copy(doc_tabt_hbm.at[:, pl.ds(0, _BLK)],
                                  blocks_v.at[slot], dsems[slot]).wait()
            idx = dvecs[d // _LANES][d % _LANES]
            lane = jnp.broadcast_to(idx & 127, (_LANES,))
            for fc in range(_D // _LANES):
                rows = lax.iota(jnp.int32, _LANES) + fc * _LANES
                vals = plsc.load_gather(blocks_v.at[slot], [rows, lane])
                drows_v[d, pl.ds(fc * _LANES, _LANES)] = vals

        for d in range(_NSLOT):
            enq_doc(d)

        def enq_word(c, carry):
            vec = widx_v[pl.ds(c * _LANES, _LANES)]
            for k in range(_LANES):
                pltpu.async_copy(word_tab_hbm.at[pl.ds(vec[k], 1)],
                                 wrows_v.at[pl.ds(c * _LANES + k, 1)], wsem)
            return carry

        lax.fori_loop(0, n_widx // _LANES, enq_word, 0)

        for d in range(_NSLOT, bpw):
            extract_doc(d - _NSLOT)
            enq_doc(d)
        for d in range(bpw - _NSLOT, bpw):
            extract_doc(d)

        pltpu.make_async_copy(word_tab_hbm.at[pl.ds(0, n_widx)], wrows_v,
                              wsem).wait()

        scale = jnp.float32(1.0 / (_CTX + 1))

        def body(i, carry):
            for c in range(_D // _LANES):
                sl = pl.ds(c * _LANES, _LANES)
                acc = drows_v[i, sl]
                for j in range(_CTX):
                    acc = acc + wrows_v[i * _CTX + j, sl]
                orows_v[i, sl] = acc * scale
            return carry

        lax.fori_loop(0, bpw, body, 0)
        pltpu.sync_copy(orows_v, out_hbm.at[pl.ds(base, bpw)])

    return sc_mean, nw


def _mm_body_t(wt_ref, meant_ref, b_ref, out_ref):
    acc = jax.lax.dot_general(
        wt_ref[...], meant_ref[...], (((0,), (0,)), ((), ())),
        preferred_element_type=jnp.float32,
    )
    out_ref[...] = acc + jnp.transpose(b_ref[...])


def _project_t(meant, Wt, b, tile_v: int = 4096):
    v = Wt.shape[1]
    grid = (pl.cdiv(v, tile_v),)
    return pl.pallas_call(
        _mm_body_t,
        grid=grid,
        in_specs=[
            pl.BlockSpec((_D, tile_v), lambda j: (0, j)),
            pl.BlockSpec((_D, _B), lambda j: (0, 0)),
            pl.BlockSpec((1, tile_v), lambda j: (0, j)),
        ],
        out_specs=pl.BlockSpec((tile_v, _B), lambda j: (j, 0)),
        out_shape=jax.ShapeDtypeStruct((v, _B), jnp.float32),
        compiler_params=pltpu.CompilerParams(
            dimension_semantics=("parallel",),
        ),
    )(Wt, meant, b.reshape(1, v))


def kernel(doc_ids, context_words, doc_table, word_table, W, b):
    info = plsc.get_sparse_core_info()
    sc_mean, nw = _build_sc_mean(info.num_cores, info.num_subcores)
    doc_ids32 = doc_ids.astype(jnp.int32)
    words = context_words.astype(jnp.int32).reshape(-1)
    mean_vec = sc_mean(doc_ids32, words, doc_table.T, word_table)
    return _project_t(mean_vec.T, W.T, b).T
